# Initial kernel scaffold; baseline (speedup 1.0000x reference)
import jax
import jax.numpy as jnp
from jax.experimental import pallas as pl


def kernel(x, W1, b1, W2, b2):
    raise NotImplementedError("write your pallas kernel here")


# speedup vs baseline: 1.0001x; 1.0001x over previous
import functools

import jax
import jax.numpy as jnp
from jax.experimental import pallas as pl
from jax.experimental.pallas import tpu as pltpu

B, S, D = 4, 8192, 2048
H = D // 2
E = 64
S_BLK = 512


def _router_kernel(x_ref, w1_ref, b1_ref, w2_ref, b2_ref, out_ref, acc_ref):
    i = pl.program_id(0)
    n = pl.num_programs(0)

    part = jnp.sum(x_ref[...], axis=1)

    @pl.when(i == 0)
    def _init():
        acc_ref[...] = part

    @pl.when(i > 0)
    def _accum():
        acc_ref[...] += part

    @pl.when(i == n - 1)
    def _finish():
        pooled = acc_ref[...] * (1.0 / S)
        h = jax.lax.dot_general(
            pooled, w1_ref[...], (((1,), (0,)), ((), ())),
            preferred_element_type=jnp.float32,
        ) + b1_ref[...]
        h = 0.5 * h * (1.0 + jax.lax.erf(h * (2.0 ** -0.5)))
        logits = jax.lax.dot_general(
            h, w2_ref[...], (((1,), (0,)), ((), ())),
            preferred_element_type=jnp.float32,
        ) + b2_ref[...]
        m = jnp.max(logits, axis=-1, keepdims=True)
        e = jnp.exp(logits - m)
        out_ref[...] = e / jnp.sum(e, axis=-1, keepdims=True)


@jax.jit
def kernel(x, W1, b1, W2, b2):
    n_blk = S // S_BLK
    grid = (n_blk,)
    out = pl.pallas_call(
        _router_kernel,
        grid=grid,
        in_specs=[
            pl.BlockSpec((B, S_BLK, D), lambda i: (0, i, 0)),
            pl.BlockSpec((D, H), lambda i: (0, 0)),
            pl.BlockSpec((1, H), lambda i: (0, 0)),
            pl.BlockSpec((H, E), lambda i: (0, 0)),
            pl.BlockSpec((1, E), lambda i: (0, 0)),
        ],
        out_specs=pl.BlockSpec((B, E), lambda i: (0, 0)),
        out_shape=jax.ShapeDtypeStruct((B, E), jnp.float32),
        scratch_shapes=[pltpu.VMEM((B, D), jnp.float32)],
        compiler_params=pltpu.CompilerParams(
            dimension_semantics=("arbitrary",),
        ),
    )(x, W1, b1.reshape(1, H), W2, b2.reshape(1, E))
    return out
